# Initial kernel scaffold; baseline (speedup 1.0000x reference)
import jax
import jax.numpy as jnp
from jax.experimental import pallas as pl


def kernel(class_logits, box_regression, proposals):
    raise NotImplementedError("write your pallas kernel here")


# speedup vs baseline: 29.4972x; 29.4972x over previous
import functools

import jax
import jax.numpy as jnp
from jax.experimental import pallas as pl

N = 20000
NUM_CLASSES = 2
IMG_H, IMG_W = 800.0, 1333.0
SCORE_THRESH = 0.05
NMS_THRESH = 0.5
DETS_PER_IMG = 100
PRE_NMS_K = 1000
BBOX_XFORM_CLIP = 4.135166556742356


def _decode_filter_body(logits_ref, reg_ref, prop_ref, boxes_ref, score_ref):
    l0 = logits_ref[0:1, :]
    l1 = logits_ref[1:2, :]
    m = jnp.maximum(l0, l1)
    e0 = jnp.exp(l0 - m)
    e1 = jnp.exp(l1 - m)
    score = e1 / (e0 + e1)

    px1 = prop_ref[0:1, :]
    py1 = prop_ref[1:2, :]
    px2 = prop_ref[2:3, :]
    py2 = prop_ref[3:4, :]
    widths = px2 - px1
    heights = py2 - py1
    ctr_x = px1 + 0.5 * widths
    ctr_y = py1 + 0.5 * heights
    dx = reg_ref[4:5, :] / 10.0
    dy = reg_ref[5:6, :] / 10.0
    dw = jnp.minimum(reg_ref[6:7, :] / 5.0, BBOX_XFORM_CLIP)
    dh = jnp.minimum(reg_ref[7:8, :] / 5.0, BBOX_XFORM_CLIP)
    pred_ctr_x = dx * widths + ctr_x
    pred_ctr_y = dy * heights + ctr_y
    pred_w = jnp.exp(dw) * widths
    pred_h = jnp.exp(dh) * heights
    x1 = pred_ctr_x - 0.5 * pred_w
    y1 = pred_ctr_y - 0.5 * pred_h
    x2 = pred_ctr_x + 0.5 * pred_w
    y2 = pred_ctr_y + 0.5 * pred_h
    x1 = jnp.minimum(jnp.maximum(x1, 0.0), IMG_W)
    y1 = jnp.minimum(jnp.maximum(y1, 0.0), IMG_H)
    x2 = jnp.minimum(jnp.maximum(x2, 0.0), IMG_W)
    y2 = jnp.minimum(jnp.maximum(y2, 0.0), IMG_H)
    ws = x2 - x1
    hs = y2 - y1
    valid = (score > SCORE_THRESH) & (ws >= 0.01) & (hs >= 0.01)
    boxes_ref[0:1, :] = x1
    boxes_ref[1:2, :] = y1
    boxes_ref[2:3, :] = x2
    boxes_ref[3:4, :] = y2
    score_ref[...] = jnp.where(valid, score, -1.0)


def _nms_body(boxes_ref, boxesT_ref, scores_ref, out_ref):
    K = PRE_NMS_K
    bx = boxes_ref[...]
    off = jnp.max(bx) + 1.0

    cx1 = bx[:, 0:1] + off
    cy1 = bx[:, 1:2] + off
    cx2 = bx[:, 2:3] + off
    cy2 = bx[:, 3:4] + off
    rx1 = boxesT_ref[0:1, :] + off
    ry1 = boxesT_ref[1:2, :] + off
    rx2 = boxesT_ref[2:3, :] + off
    ry2 = boxesT_ref[3:4, :] + off
    area_c = (cx2 - cx1) * (cy2 - cy1)
    area_r = (rx2 - rx1) * (ry2 - ry1)

    ltx = jnp.maximum(cx1, rx1)
    lty = jnp.maximum(cy1, ry1)
    rbx = jnp.minimum(cx2, rx2)
    rby = jnp.minimum(cy2, ry2)
    iw = jnp.maximum(rbx - ltx, 0.0)
    ih = jnp.maximum(rby - lty, 0.0)
    inter = iw * ih
    iou = inter / (area_c + area_r - inter + 1e-9)

    row = jax.lax.broadcasted_iota(jnp.int32, (K, K), 0)
    col = jax.lax.broadcasted_iota(jnp.int32, (K, K), 1)
    hit = (iou > NMS_THRESH).astype(jnp.float32)
    upper = jnp.where(row < col, hit, 0.0)
    lower = jnp.where(col < row, hit, 0.0)

    def fc(k_row):
        susp = jnp.max(lower * k_row, axis=1, keepdims=True)
        return 1.0 - susp

    def fr(k_col):
        susp = jnp.max(upper * k_col, axis=0, keepdims=True)
        return 1.0 - susp

    ones_row = jnp.ones((1, K), dtype=jnp.float32)

    def cond(state):
        low, up, it = state
        return jnp.logical_and(it < K, jnp.any(low != up))

    def body(state):
        low, up, it = state
        return fr(fc(low)), fr(fc(up)), it + 2

    low0 = fr(fc(jnp.zeros((1, K), dtype=jnp.float32)))
    up0 = fr(fc(ones_row))
    low, up, _ = jax.lax.while_loop(cond, body, (low0, up0, jnp.int32(2)))

    sc = scores_ref[...]
    keep = (low > 0.0) & (sc > 0.0)
    out_ref[...] = jnp.where(keep, sc, -1.0)


@jax.jit
def kernel(class_logits, box_regression, proposals):
    logits_t = class_logits.T
    reg_t = box_regression.T
    prop_t = proposals.T

    boxes_t, masked = pl.pallas_call(
        _decode_filter_body,
        out_shape=(
            jax.ShapeDtypeStruct((4, N), jnp.float32),
            jax.ShapeDtypeStruct((1, N), jnp.float32),
        ),
    )(logits_t, reg_t, prop_t)

    top_scores, top_idx = jax.lax.top_k(masked[0], PRE_NMS_K)
    top_boxes = boxes_t[:, top_idx]

    final = pl.pallas_call(
        _nms_body,
        out_shape=jax.ShapeDtypeStruct((1, PRE_NMS_K), jnp.float32),
    )(top_boxes.T, top_boxes, top_scores[None, :])

    det_scores, det_idx = jax.lax.top_k(final[0], DETS_PER_IMG)
    det_boxes = top_boxes.T[det_idx]
    det_labels = jnp.ones((DETS_PER_IMG,), dtype=jnp.int32)
    return det_boxes, det_scores, det_labels


# speedup vs baseline: 34.0302x; 1.1537x over previous
import jax
import jax.numpy as jnp
from jax.experimental import pallas as pl
from jax.experimental.pallas import tpu as pltpu

N = 20000
NP = 20480
GR = 160
GC = 128
NCHUNK = 20
IMG_H, IMG_W = 800.0, 1333.0
SCORE_THRESH = 0.05
NMS_THRESH = 0.5
DETS = 100
K = 1000
BBOX_XFORM_CLIP = 4.135166556742356


def _decode(l0, l1, r4, r5, r6, r7, px1, py1, px2, py2):
    m = jnp.maximum(l0, l1)
    e0 = jnp.exp(l0 - m)
    e1 = jnp.exp(l1 - m)
    score = e1 / (e0 + e1)
    widths = px2 - px1
    heights = py2 - py1
    ctr_x = px1 + 0.5 * widths
    ctr_y = py1 + 0.5 * heights
    dx = r4 / 10.0
    dy = r5 / 10.0
    dw = jnp.minimum(r6 / 5.0, BBOX_XFORM_CLIP)
    dh = jnp.minimum(r7 / 5.0, BBOX_XFORM_CLIP)
    pred_ctr_x = dx * widths + ctr_x
    pred_ctr_y = dy * heights + ctr_y
    pred_w = jnp.exp(dw) * widths
    pred_h = jnp.exp(dh) * heights
    x1 = pred_ctr_x - 0.5 * pred_w
    y1 = pred_ctr_y - 0.5 * pred_h
    x2 = pred_ctr_x + 0.5 * pred_w
    y2 = pred_ctr_y + 0.5 * pred_h
    x1 = jnp.minimum(jnp.maximum(x1, 0.0), IMG_W)
    y1 = jnp.minimum(jnp.maximum(y1, 0.0), IMG_H)
    x2 = jnp.minimum(jnp.maximum(x2, 0.0), IMG_W)
    y2 = jnp.minimum(jnp.maximum(y2, 0.0), IMG_H)
    valid = (score > SCORE_THRESH) & ((x2 - x1) >= 0.01) & ((y2 - y1) >= 0.01)
    ms = jnp.where(valid, score, -1.0)
    return ms, x1, y1, x2, y2


def _iota(shape, dim, dtype=jnp.int32):
    return jax.lax.broadcasted_iota(dtype, shape, dim)


def _body(g_ref, f_ref, det_ref, vals_ref, pos_ref):
    g = [g_ref[i] for i in range(10)]
    ms_g, _, _, _, _ = _decode(*g)
    bits = jax.lax.bitcast_convert_type(ms_g, jnp.int32)
    key = jnp.where(bits >= 0, bits, bits ^ jnp.int32(0x7FFFFFFF))

    f = [f_ref[i:i + 1, :] for i in range(10)]
    flds = _decode(*f)
    for fi in range(5):
        b = jax.lax.bitcast_convert_type(flds[fi], jnp.int32)
        for pi in range(4):
            plane = (b >> (8 * pi)) & 255
            vals_ref[4 * fi + pi:4 * fi + pi + 1, :] = plane.astype(jnp.bfloat16)

    lo0 = jnp.min(key) - 1
    hi0 = jnp.max(key)

    def vbody(_, c):
        lo, hi = c
        mid = lo + (hi - lo) // 2
        cnt = jnp.sum(jnp.where(key > mid, 1, 0))
        small = cnt < K
        return jnp.where(small, lo, mid), jnp.where(small, mid, hi)

    _, v = jax.lax.fori_loop(0, 31, vbody, (lo0, hi0))
    n_above = jnp.sum(jnp.where(key > v, 1, 0))
    need = K - n_above

    idx = _iota((GR, GC), 0) * GC + _iota((GR, GC), 1)
    tie = key == v

    def mbody2(_, c):
        lo, hi = c
        mid = lo + (hi - lo) // 2
        cnt = jnp.sum(jnp.where(tie & (idx < mid), 1, 0))
        big = cnt >= need
        return jnp.where(big, lo, mid), jnp.where(big, mid, hi)

    _, mstar = jax.lax.fori_loop(0, 15, mbody2, (jnp.int32(0), jnp.int32(NP)))
    sel = (key > v) | (tie & (idx < mstar))

    self_f = sel.astype(jnp.float32)
    u128 = (_iota((GC, GC), 0) < _iota((GC, GC), 1)).astype(jnp.float32)
    cum = jax.lax.dot_general(self_f, u128, (((1,), (0,)), ((), ())),
                              preferred_element_type=jnp.float32, precision=jax.lax.Precision.HIGHEST)
    row_tot = cum[:, GC - 1:GC] + self_f[:, GC - 1:GC]
    t160 = (_iota((GR, GR), 1) < _iota((GR, GR), 0)).astype(jnp.float32)
    offs = jax.lax.dot_general(t160, row_tot, (((1,), (0,)), ((), ())),
                               preferred_element_type=jnp.float32, precision=jax.lax.Precision.HIGHEST)
    pos = cum + offs
    pos_ref[...] = jnp.where(sel, pos, 2000.0)

    sub1024 = _iota((1024, GC), 0)
    lane128 = _iota((1024, GC), 1)
    colmask = lane128 == (sub1024 & 127)
    p_lane1024 = _iota((1, 1024), 1).astype(jnp.float32)

    def cbody(c, acc):
        pch = pos_ref[pl.ds(c * 8, 8), :]
        bc = jnp.broadcast_to(pch[:, None, :], (8, GC, GC)).reshape(1024, GC)
        pcol = jnp.sum(jnp.where(colmask, bc, 0.0), axis=1, keepdims=True)
        onehot = (pcol == p_lane1024).astype(jnp.bfloat16)
        vch = vals_ref[:, pl.ds(c * 1024, 1024)]
        return acc + jax.lax.dot_general(
            vch, onehot, (((1,), (0,)), ((), ())),
            preferred_element_type=jnp.float32)

    planes_c = jax.lax.fori_loop(
        0, NCHUNK, cbody, jnp.zeros((20, 1024), jnp.float32))

    def _reassemble(planes):
        rows = []
        for fi in range(5):
            w = planes[4 * fi:4 * fi + 1, :].astype(jnp.int32)
            for pi in range(1, 4):
                w = w | (planes[4 * fi + pi:4 * fi + pi + 1, :].astype(jnp.int32)
                         << (8 * pi))
            rows.append(jax.lax.bitcast_convert_type(w, jnp.float32))
        return jnp.concatenate(rows, axis=0)

    vals_c = _reassemble(planes_c)

    lane_lt_sub = _iota((1024, 1024), 1) < _iota((1024, 1024), 0)
    rank_lane = _iota((1024, 1024), 1).astype(jnp.float32)

    s_row = jnp.where(_iota((1, 1024), 1) < K, vals_c[0:1, :], -2.0)
    s_col = jnp.transpose(s_row)
    gt = (s_row > s_col) | ((s_row == s_col) & lane_lt_sub)
    rank_col = jnp.sum(gt.astype(jnp.float32), axis=1, keepdims=True)
    perm = (rank_col == rank_lane).astype(jnp.bfloat16)
    planes_s = jax.lax.dot_general(planes_c.astype(jnp.bfloat16), perm,
                                   (((1,), (0,)), ((), ())),
                                   preferred_element_type=jnp.float32)
    sorted_v = _reassemble(planes_s)
    sorted_t = jnp.transpose(sorted_v)

    sc_row = sorted_v[0:1, 0:K]
    off = jnp.max(sorted_v[1:5, 0:K]) + 1.0
    rx1 = sorted_v[1:2, 0:K] + off
    ry1 = sorted_v[2:3, 0:K] + off
    rx2 = sorted_v[3:4, 0:K] + off
    ry2 = sorted_v[4:5, 0:K] + off
    cx1 = sorted_t[0:K, 1:2] + off
    cy1 = sorted_t[0:K, 2:3] + off
    cx2 = sorted_t[0:K, 3:4] + off
    cy2 = sorted_t[0:K, 4:5] + off
    area_r = (rx2 - rx1) * (ry2 - ry1)
    area_c = (cx2 - cx1) * (cy2 - cy1)
    iw = jnp.maximum(jnp.minimum(cx2, rx2) - jnp.maximum(cx1, rx1), 0.0)
    ih = jnp.maximum(jnp.minimum(cy2, ry2) - jnp.maximum(cy1, ry1), 0.0)
    inter = iw * ih
    iou = inter / (area_c + area_r - inter + 1e-9)

    row_i = _iota((K, K), 0)
    col_i = _iota((K, K), 1)
    hit = (iou > NMS_THRESH).astype(jnp.bfloat16)
    upper = jnp.where(row_i < col_i, hit, jnp.bfloat16(0))

    def f2(s):
        cnt = jax.lax.dot_general(s.astype(jnp.bfloat16), upper,
                                  (((1,), (0,)), ((), ())),
                                  preferred_element_type=jnp.float32)
        return (cnt < 0.5).astype(jnp.float32)

    def cond(st):
        s, it = st
        return jnp.logical_and(it < K, jnp.any(s[0:1, :] != s[1:2, :]))

    def wbody(st):
        s, it = st
        r = f2(s)
        return jnp.concatenate([r[1:2, :], r[0:1, :]], axis=0), it + 1

    s0 = jnp.concatenate([jnp.zeros((1, K), jnp.float32),
                          jnp.ones((1, K), jnp.float32)], axis=0)
    s_fin, _ = jax.lax.while_loop(cond, wbody, (s0, jnp.int32(0)))
    keep_row = s_fin[0:1, :]
    keep_col = jnp.transpose(keep_row)

    f_row = jnp.where((keep_row > 0.0) & (sc_row > 0.0), sc_row, -1.0)
    sc_col = sorted_t[0:K, 0:1]
    f_col = jnp.where((keep_col > 0.0) & (sc_col > 0.0), sc_col, -1.0)
    gt2 = (f_row > f_col) | ((f_row == f_col) & (_iota((K, K), 1) < _iota((K, K), 0)))
    drank_col = jnp.sum(gt2.astype(jnp.float32), axis=1, keepdims=True)
    pdet = (drank_col == _iota((K, 128), 1).astype(jnp.float32)).astype(jnp.float32)
    det = jax.lax.dot_general(sorted_v[:, 0:K], pdet, (((1,), (0,)), ((), ())),
                              preferred_element_type=jnp.float32, precision=jax.lax.Precision.HIGHEST)
    det_ref[...] = det


@jax.jit
def kernel(class_logits, box_regression, proposals):
    flat = jnp.concatenate(
        [class_logits.T, box_regression[:, 4:8].T, proposals.T], axis=0)
    flat = jnp.pad(flat, ((0, 0), (0, NP - N)))
    grid = flat.reshape(10, GR, GC)

    det = pl.pallas_call(
        _body,
        out_shape=jax.ShapeDtypeStruct((5, 128), jnp.float32),
        scratch_shapes=[
            pltpu.VMEM((20, NP), jnp.bfloat16),
            pltpu.VMEM((GR, GC), jnp.float32),
        ],
    )(grid, flat)

    det_boxes = det[1:5, 0:DETS].T
    det_scores = det[0, 0:DETS]
    det_labels = jnp.ones((DETS,), dtype=jnp.int32)
    return det_boxes, det_scores, det_labels
